# baseline (speedup 1.0000x reference)
import functools

import jax
import jax.numpy as jnp
from jax import lax
from jax.experimental import pallas as pl
from jax.experimental.pallas import tpu as pltpu
from jax.experimental.pallas import tpu_sc as plsc

_B = 16384
_C = 1000
_NW = 32
_BPW = _B // _NW
_CROWS = 32
_NCHUNK = -(-_C // _CROWS)
_L = 16
_TAB = 128

_mesh = plsc.VectorSubcoreMesh(core_axis_name="c", subcore_axis_name="s")


@functools.partial(
    pl.kernel,
    out_type=jax.ShapeDtypeStruct((_C, _B), jnp.float32),
    mesh=_mesh,
    scratch_types=[
        pltpu.VMEM((_BPW,), jnp.int32),
        pltpu.VMEM((_TAB,), jnp.int32),
        pltpu.VMEM((_CROWS, _BPW), jnp.float32),
        pltpu.VMEM((_CROWS, _BPW), jnp.float32),
        pltpu.SemaphoreType.DMA,
        pltpu.SemaphoreType.DMA,
    ],
    compiler_params=pltpu.CompilerParams(
        needs_layout_passes=False, use_tc_tiling_on_sc=True
    ),
)
def _onehot_sc(cond_hbm, tab_hbm, out_hbm, cond_v, tab_v, buf0, buf1, sem0, sem1):
    wid = lax.axis_index("s") * 2 + lax.axis_index("c")
    base = wid * _BPW

    pltpu.sync_copy(cond_hbm.at[pl.ds(base, _BPW)], cond_v)
    pltpu.sync_copy(tab_hbm, tab_v)

    zeros = jnp.zeros((_L,), jnp.float32)
    ones = jnp.ones((_L,), jnp.float32)
    iota = lax.iota(jnp.int32, _L)

    def _zero_buf(buf):
        def body(r, _):
            for s in range(_BPW // _L):
                buf[r, pl.ds(s * _L, _L)] = zeros
            return 0

        lax.fori_loop(0, _CROWS, body, 0)

    bufs = (buf0, buf1)
    sems = (sem0, sem1)
    copies = [None, None]

    def _scatter_chunk(t, buf, val):
        c0 = t * _CROWS
        rows = min(_C - c0, _CROWS)

        def body(j, _):
            cond16 = cond_v[pl.ds(j * _L, _L)]
            code16 = plsc.load_gather(tab_v, [cond16])
            crow16 = code16 - c0
            mask = (crow16 >= 0) & (crow16 < rows)
            crow16 = jnp.where(mask, crow16, 0)
            plsc.store_scatter(buf, [crow16, iota + j * _L], val, mask=mask)
            return 0

        lax.fori_loop(0, _BPW // _L, body, 0)

    for t in range(_NCHUNK):
        b = t % 2
        buf = bufs[b]
        if t < 2:
            _zero_buf(buf)
        else:
            copies[b].wait()
            _scatter_chunk(t - 2, buf, zeros)
        _scatter_chunk(t, buf, ones)
        rows = min(_C - t * _CROWS, _CROWS)
        dst = out_hbm.at[pl.ds(t * _CROWS, rows), pl.ds(base, _BPW)]
        copies[b] = pltpu.async_copy(buf.at[pl.ds(0, rows)], dst, sems[b])

    copies[0].wait()
    copies[1].wait()


def kernel(condition, celltype_codes):
    tab = jnp.zeros((_TAB,), jnp.int32).at[: celltype_codes.shape[0]].set(
        celltype_codes
    )
    return _onehot_sc(condition, tab).T


# speedup vs baseline: 1.2969x; 1.2969x over previous
import functools

import jax
import jax.numpy as jnp
from jax import lax
from jax.experimental import pallas as pl
from jax.experimental.pallas import tpu as pltpu
from jax.experimental.pallas import tpu_sc as plsc

_B = 16384
_C = 1000
_NW = 32
_BPW = _B // _NW
_CROWS = 80
_NCHUNK = -(-_C // _CROWS)
_L = 16
_TAB = 128

_mesh = plsc.VectorSubcoreMesh(core_axis_name="c", subcore_axis_name="s")


@functools.partial(
    pl.kernel,
    out_type=jax.ShapeDtypeStruct((_C, _B), jnp.float32),
    mesh=_mesh,
    scratch_types=[
        pltpu.VMEM((_BPW,), jnp.int32),
        pltpu.VMEM((_TAB,), jnp.int32),
        pltpu.VMEM((_CROWS, _BPW), jnp.float32),
        pltpu.VMEM((_CROWS, _BPW), jnp.float32),
        pltpu.SemaphoreType.DMA,
        pltpu.SemaphoreType.DMA,
    ],
    compiler_params=pltpu.CompilerParams(
        needs_layout_passes=False, use_tc_tiling_on_sc=True
    ),
)
def _onehot_sc(cond_hbm, tab_hbm, out_hbm, cond_v, tab_v, buf0, buf1, sem0, sem1):
    wid = lax.axis_index("s") * 2 + lax.axis_index("c")
    base = wid * _BPW

    pltpu.sync_copy(cond_hbm.at[pl.ds(base, _BPW)], cond_v)
    pltpu.sync_copy(tab_hbm, tab_v)

    zeros = jnp.zeros((_L,), jnp.float32)
    ones = jnp.ones((_L,), jnp.float32)
    iota = lax.iota(jnp.int32, _L)

    def _zero_buf(buf):
        def body(r, _):
            for s in range(_BPW // _L):
                buf[r, pl.ds(s * _L, _L)] = zeros
            return 0

        lax.fori_loop(0, _CROWS, body, 0)

    bufs = (buf0, buf1)
    sems = (sem0, sem1)
    copies = [None, None]

    def _scatter_chunk(t, buf, val):
        c0 = t * _CROWS
        rows = min(_C - c0, _CROWS)

        def body(j, _):
            cond16 = cond_v[pl.ds(j * _L, _L)]
            code16 = plsc.load_gather(tab_v, [cond16])
            crow16 = code16 - c0
            mask = (crow16 >= 0) & (crow16 < rows)
            crow16 = jnp.where(mask, crow16, 0)
            plsc.store_scatter(buf, [crow16, iota + j * _L], val, mask=mask)
            return 0

        lax.fori_loop(0, _BPW // _L, body, 0)

    for t in range(_NCHUNK):
        b = t % 2
        buf = bufs[b]
        if t < 2:
            _zero_buf(buf)
        else:
            copies[b].wait()
            _scatter_chunk(t - 2, buf, zeros)
        _scatter_chunk(t, buf, ones)
        rows = min(_C - t * _CROWS, _CROWS)
        dst = out_hbm.at[pl.ds(t * _CROWS, rows), pl.ds(base, _BPW)]
        copies[b] = pltpu.async_copy(buf.at[pl.ds(0, rows)], dst, sems[b])

    copies[0].wait()
    copies[1].wait()


def kernel(condition, celltype_codes):
    tab = jnp.zeros((_TAB,), jnp.int32).at[: celltype_codes.shape[0]].set(
        celltype_codes
    )
    return _onehot_sc(condition, tab).T
